# Initial kernel scaffold; baseline (speedup 1.0000x reference)
import jax
import jax.numpy as jnp
from jax.experimental import pallas as pl


def kernel(x_src, x_dst, edge_index):
    raise NotImplementedError("write your pallas kernel here")


# speedup vs baseline: 2.2764x; 2.2764x over previous
import functools

import jax
import jax.numpy as jnp
from jax import lax
from jax.experimental import pallas as pl
from jax.experimental.pallas import tpu as pltpu
from jax.experimental.pallas import tpu_sc as plsc

N_SRC = 10000
N_DST = 10000
E = 320000
D = 128

NC = 2
NS = 16
L = 16
NW = NC * NS
RPT = 320
TRASH = RPT
ROWS = RPT + L
CH = 3200
NCH = E // CH
FG = D // L
DUMP = CH + L


def _sc_body(xs, xd, srci, dsti, out,
             dstbuf, srcbuf, midx, msrc, asum, amax, acnt,
             rowbuf, meanbuf, maxbuf, xdbuf, sem):
  wid = lax.axis_index("s") * NC + lax.axis_index("c")
  lo = wid * RPT
  nrows = jnp.minimum(N_DST - lo, RPT)
  lane = lax.iota(jnp.int32, L)
  zi = jnp.zeros((L,), jnp.int32)
  zf = jnp.zeros((L,), jnp.float32)
  ninf = jnp.full((L,), -jnp.inf, jnp.float32)
  ones = jnp.ones((L,), jnp.float32)

  def init_row(r, c):
    for f in range(FG):
      asum[r, pl.ds(f * L, L)] = zf
      amax[r, pl.ds(f * L, L)] = ninf
    return c
  lax.fori_loop(0, ROWS, init_row, 0)

  def init_cnt(i, c):
    acnt[pl.ds(i * L, L)] = zf
    return c
  lax.fori_loop(0, ROWS // L, init_cnt, 0)

  def init_msrc(i, c):
    msrc[pl.ds(i * L, L)] = zi
    return c
  lax.fori_loop(0, (CH + 2 * L) // L, init_msrc, 0)

  def chunk_body(c, carry):
    pltpu.sync_copy(dsti.at[pl.ds(c * CH, CH)], dstbuf)
    pltpu.sync_copy(srci.at[pl.ds(c * CH, CH)], srcbuf)

    def scan_body(i, n):
      dvec = dstbuf[pl.ds(i * L, L)]
      svec = srcbuf[pl.ds(i * L, L)]
      m = (dvec >= lo) & (dvec < lo + nrows)
      inc = m.astype(jnp.int32)
      pos = jnp.where(m, n + plsc.cumsum(inc) - 1, DUMP + lane)
      plsc.store_scatter(midx, [pos], dvec - lo)
      plsc.store_scatter(msrc, [pos], svec)
      return n + jnp.sum(inc)

    n = lax.fori_loop(0, CH // L, scan_body, jnp.int32(0))
    midx[pl.ds(n, L)] = jnp.full((L,), TRASH, jnp.int32)

    def group_body(g, carry2):
      sidx = msrc[pl.ds(g * L, L)]
      pltpu.async_copy(xs.at[sidx], rowbuf, sem).wait()
      dvec = midx[pl.ds(g * L, L)]
      plsc.addupdate_scatter(acnt, [dvec], ones)
      for j in range(L):
        d = dvec[j]
        for f in range(FG):
          v = rowbuf[j, pl.ds(f * L, L)]
          plsc.addupdate(asum.at[d, pl.ds(f * L, L)], v)
          amax[d, pl.ds(f * L, L)] = jnp.maximum(amax[d, pl.ds(f * L, L)], v)
      return carry2

    lax.fori_loop(0, (n + L - 1) // L, group_body, 0)
    return carry

  lax.fori_loop(0, NCH, chunk_body, 0)

  def fin_body(b, carry):
    r0 = b * L
    cvec = acnt[pl.ds(r0, L)]
    rvec = 1.0 / jnp.maximum(cvec, 1.0)
    pltpu.sync_copy(xd.at[pl.ds(lo + r0, L)], xdbuf)
    for j in range(L):
      cj = cvec[j]
      rj = rvec[j]
      for f in range(FG):
        s = asum[r0 + j, pl.ds(f * L, L)]
        meanbuf[j, pl.ds(f * L, L)] = s * rj
        mx = amax[r0 + j, pl.ds(f * L, L)]
        maxbuf[j, pl.ds(f * L, L)] = jnp.where(cj > 0.0, mx, zf)
    pltpu.sync_copy(xdbuf, out.at[pl.ds(lo + r0, L), pl.ds(0, D)])
    pltpu.sync_copy(meanbuf, out.at[pl.ds(lo + r0, L), pl.ds(D, D)])
    pltpu.sync_copy(maxbuf, out.at[pl.ds(lo + r0, L), pl.ds(2 * D, D)])
    return carry

  lax.fori_loop(0, nrows // L, fin_body, 0)


_sc_kernel = functools.partial(
    pl.kernel,
    out_type=jax.ShapeDtypeStruct((N_DST, 3 * D), jnp.float32),
    mesh=plsc.VectorSubcoreMesh(
        core_axis_name="c", subcore_axis_name="s",
        num_cores=NC, num_subcores=NS),
    compiler_params=pltpu.CompilerParams(needs_layout_passes=False),
    scratch_types=[
        pltpu.VMEM((CH,), jnp.int32),
        pltpu.VMEM((CH,), jnp.int32),
        pltpu.VMEM((CH + 2 * L,), jnp.int32),
        pltpu.VMEM((CH + 2 * L,), jnp.int32),
        pltpu.VMEM((ROWS, D), jnp.float32),
        pltpu.VMEM((ROWS, D), jnp.float32),
        pltpu.VMEM((ROWS,), jnp.float32),
        pltpu.VMEM((L, D), jnp.float32),
        pltpu.VMEM((L, D), jnp.float32),
        pltpu.VMEM((L, D), jnp.float32),
        pltpu.VMEM((L, D), jnp.float32),
        pltpu.SemaphoreType.DMA,
    ],
)(_sc_body)


def kernel(x_src, x_dst, edge_index):
  return _sc_kernel(x_src, x_dst, edge_index[0], edge_index[1])
